# baseline (speedup 1.0000x reference)
import functools

import jax
import jax.numpy as jnp
from jax import lax
from jax.experimental import pallas as pl
from jax.experimental.pallas import tpu as pltpu
from jax.experimental.pallas import tpu_sc as plsc

BATCH = 16384
HIST = 200
VPAD = 128
NUM_WORKERS = 32
ROWS_PER_W = BATCH // NUM_WORKERS
WROWS = BATCH // 2
WROWS_PER_W = ROWS_PER_W // 2


NBUF = 4
WORDS_PER_W = WROWS_PER_W * 128
CHUNK = 128


def _hist_body(x_hbm, counts_hbm, x_v, wide_v, *bufs):
    nc = 2
    wid = lax.axis_index("s") * nc + lax.axis_index("c")

    pltpu.sync_copy(x_hbm.at[pl.ds(wid * WORDS_PER_W, WORDS_PER_W)], x_v)

    zeros = jnp.zeros((16,), jnp.float32)
    izeros = jnp.zeros((16,), jnp.int32)
    iota16 = lax.iota(jnp.int32, 16)
    iones = jnp.ones((16,), jnp.int32)

    @plsc.parallel_loop(0, ROWS_PER_W)
    def _zero(i):
        for b in bufs:
            b[i, pl.ds(0, 16)] = izeros
            b[i, pl.ds(8, 16)] = izeros

    @plsc.parallel_loop(0, CHUNK)
    def _zero_wide(i):
        for c in (24, 40, 56, 72, 88, 104, 112):
            wide_v[i, pl.ds(c, 16)] = zeros

    @plsc.parallel_loop(0, WROWS_PER_W // 16)
    def _groups(jg):
        r_lo = iota16 + jg * 16
        wrows128 = r_lo * 128
        r_hi = r_lo + WROWS_PER_W

        @plsc.parallel_loop(0, 72, unroll=2)
        def _hist_l(l):
            w = plsc.load_gather(x_v, [wrows128 + l])
            ids0 = w & 0xFF
            ids1 = lax.shift_right_logical(w, 8) & 0xFF
            ids2 = lax.shift_right_logical(w, 16) & 0xFF
            ids3 = lax.shift_right_logical(w, 24)
            plsc.addupdate_scatter(bufs[0], [r_lo, ids0], iones)
            plsc.addupdate_scatter(bufs[1], [r_lo, ids1], iones)
            plsc.addupdate_scatter(bufs[2], [r_hi, ids2], iones)
            plsc.addupdate_scatter(bufs[3], [r_hi, ids3], iones)

        @plsc.parallel_loop(72, 128, unroll=2)
        def _hist_l2(l):
            w = plsc.load_gather(x_v, [wrows128 + l])
            ids0 = w & 0xFF
            ids2 = lax.shift_right_logical(w, 16) & 0xFF
            plsc.addupdate_scatter(bufs[0], [r_lo, ids0], iones)
            plsc.addupdate_scatter(bufs[2], [r_hi, ids2], iones)

    r_a = (wid // 4) * 2048 + (wid % 4) * WROWS_PER_W
    for c in range(ROWS_PER_W // CHUNK):
        base_local = c * CHUNK
        if c < 2:
            base_glob = r_a + base_local
        else:
            base_glob = r_a + 1024 + (base_local - 2 * CHUNK)

        @plsc.parallel_loop(0, CHUNK)
        def _merge(i):
            lo = bufs[0][base_local + i, pl.ds(0, 16)]
            hi = bufs[0][base_local + i, pl.ds(8, 16)]
            for b in bufs[1:]:
                lo = lo + b[base_local + i, pl.ds(0, 16)]
                hi = hi + b[base_local + i, pl.ds(8, 16)]
            wide_v[i, pl.ds(0, 16)] = lo.astype(jnp.float32)
            wide_v[i, pl.ds(8, 16)] = hi.astype(jnp.float32)

        pltpu.sync_copy(wide_v, counts_hbm.at[pl.ds(base_glob, CHUNK), :])


@functools.cache
def _make_hist():
    return pl.kernel(
        _hist_body,
        mesh=plsc.VectorSubcoreMesh(core_axis_name="c", subcore_axis_name="s"),
        out_type=jax.ShapeDtypeStruct((BATCH, VPAD), jnp.float32),
        scratch_types=[
            pltpu.VMEM((WORDS_PER_W,), jnp.int32),
            pltpu.VMEM((CHUNK, VPAD), jnp.float32),
        ]
        + [pltpu.VMEM((ROWS_PER_W, 24), jnp.int32) for _ in range(NBUF)],
        compiler_params=pltpu.CompilerParams(
            use_tc_tiling_on_sc=False,
            needs_layout_passes=False,
        ),
    )


def _pack_body(x_hbm, o_ref, x_v, sem):
    blk = x_v.shape[0]
    half = blk // 2
    i = pl.program_id(0)
    cp = pltpu.make_async_copy(x_hbm.at[pl.ds(i * blk, blk), :], x_v, sem)
    cp.start()
    cp.wait()
    xp = jnp.pad(x_v[:, :], ((0, 0), (0, 256 - HIST)))
    x00 = lax.slice(xp, (0, 0), (half, 128))
    x01 = lax.slice(xp, (0, 128), (half, 256))
    x10 = lax.slice(xp, (half, 0), (blk, 128))
    x11 = lax.slice(xp, (half, 128), (blk, 256))
    w = x00 | (x01 << 8) | (x10 << 16) | (x11 << 24)
    o_ref[...] = w.reshape(-1)


def _pack(x):
    blk = 2048
    return pl.pallas_call(
        _pack_body,
        grid=(BATCH // blk,),
        in_specs=[pl.BlockSpec(memory_space=pl.ANY)],
        out_specs=pl.BlockSpec((blk * 64,), lambda i: (i,)),
        out_shape=jax.ShapeDtypeStruct((BATCH * 64,), jnp.int32),
        scratch_shapes=[
            pltpu.VMEM((blk, HIST), jnp.int32),
            pltpu.SemaphoreType.DMA,
        ],
    )(x)


def _matmul_body(c_ref, t_ref, o_ref):
    o_ref[:, :] = jnp.tanh(
        jnp.dot(
            c_ref[:, :],
            t_ref[:, :],
            preferred_element_type=jnp.float32,
            precision=lax.Precision.HIGHEST,
        )
    )


def _matmul_tanh(counts, table128):
    blk = 4096
    return pl.pallas_call(
        _matmul_body,
        grid=(BATCH // blk,),
        in_specs=[
            pl.BlockSpec((blk, VPAD), lambda i: (i, 0)),
            pl.BlockSpec((VPAD, 128), lambda i: (0, 0)),
        ],
        out_specs=pl.BlockSpec((blk, 128), lambda i: (i, 0)),
        out_shape=jax.ShapeDtypeStruct((BATCH, 128), jnp.float32),
    )(counts, table128)


def kernel(x, table):
    x_pk = _pack(x)
    counts = _make_hist()(x_pk)
    table128 = jnp.concatenate([table, jnp.zeros((105, 128), table.dtype)], axis=0)
    return _matmul_tanh(counts, table128)


# speedup vs baseline: 1.1125x; 1.1125x over previous
import functools

import jax
import jax.numpy as jnp
from jax import lax
from jax.experimental import pallas as pl
from jax.experimental.pallas import tpu as pltpu
from jax.experimental.pallas import tpu_sc as plsc

BATCH = 16384
HIST = 200
VPAD = 128
NUM_WORKERS = 32
ROWS_PER_W = BATCH // NUM_WORKERS
WROWS = BATCH // 2
WROWS_PER_W = ROWS_PER_W // 2


NBUF = 4
WORDS_PER_W = WROWS_PER_W * 128
CHUNK = 128


def _hist_body(x_hbm, counts_hbm, x_v, wide_v, *bufs):
    nc = 2
    wid = lax.axis_index("s") * nc + lax.axis_index("c")

    pltpu.sync_copy(x_hbm.at[pl.ds(wid * WORDS_PER_W, WORDS_PER_W)], x_v)

    zeros = jnp.zeros((16,), jnp.float32)
    izeros = jnp.zeros((16,), jnp.int32)
    iota16 = lax.iota(jnp.int32, 16)
    iones = jnp.ones((16,), jnp.int32)

    @plsc.parallel_loop(0, ROWS_PER_W)
    def _zero(i):
        for b in bufs:
            b[i, pl.ds(0, 16)] = izeros
            b[i, pl.ds(8, 16)] = izeros

    @plsc.parallel_loop(0, CHUNK)
    def _zero_wide(i):
        for c in (24, 40, 56, 72, 88, 104, 112):
            wide_v[i, pl.ds(c, 16)] = zeros

    @plsc.parallel_loop(0, WROWS_PER_W // 16)
    def _groups(jg):
        r_lo = iota16 + jg * 16
        wrows128 = r_lo * 128
        r_hi = r_lo + WROWS_PER_W

        @plsc.parallel_loop(0, 72, unroll=4)
        def _hist_l(l):
            w = plsc.load_gather(x_v, [wrows128 + l])
            ids0 = w & 0xFF
            ids1 = lax.shift_right_logical(w, 8) & 0xFF
            ids2 = lax.shift_right_logical(w, 16) & 0xFF
            ids3 = lax.shift_right_logical(w, 24)
            plsc.addupdate_scatter(bufs[0], [r_lo, ids0], iones)
            plsc.addupdate_scatter(bufs[1], [r_lo, ids1], iones)
            plsc.addupdate_scatter(bufs[2], [r_hi, ids2], iones)
            plsc.addupdate_scatter(bufs[3], [r_hi, ids3], iones)

        @plsc.parallel_loop(72, 128, unroll=4)
        def _hist_l2(l):
            w = plsc.load_gather(x_v, [wrows128 + l])
            ids0 = w & 0xFF
            ids2 = lax.shift_right_logical(w, 16) & 0xFF
            plsc.addupdate_scatter(bufs[0], [r_lo, ids0], iones)
            plsc.addupdate_scatter(bufs[2], [r_hi, ids2], iones)

    r_a = (wid // 4) * 2048 + (wid % 4) * WROWS_PER_W
    for c in range(ROWS_PER_W // CHUNK):
        base_local = c * CHUNK
        if c < 2:
            base_glob = r_a + base_local
        else:
            base_glob = r_a + 1024 + (base_local - 2 * CHUNK)

        @plsc.parallel_loop(0, CHUNK)
        def _merge(i):
            lo = bufs[0][base_local + i, pl.ds(0, 16)]
            hi = bufs[0][base_local + i, pl.ds(8, 16)]
            for b in bufs[1:]:
                lo = lo + b[base_local + i, pl.ds(0, 16)]
                hi = hi + b[base_local + i, pl.ds(8, 16)]
            wide_v[i, pl.ds(0, 16)] = lo.astype(jnp.float32)
            wide_v[i, pl.ds(8, 16)] = hi.astype(jnp.float32)

        pltpu.sync_copy(wide_v, counts_hbm.at[pl.ds(base_glob, CHUNK), :])


@functools.cache
def _make_hist():
    return pl.kernel(
        _hist_body,
        mesh=plsc.VectorSubcoreMesh(core_axis_name="c", subcore_axis_name="s"),
        out_type=jax.ShapeDtypeStruct((BATCH, VPAD), jnp.float32),
        scratch_types=[
            pltpu.VMEM((WORDS_PER_W,), jnp.int32),
            pltpu.VMEM((CHUNK, VPAD), jnp.float32),
        ]
        + [pltpu.VMEM((ROWS_PER_W, 24), jnp.int32) for _ in range(NBUF)],
        compiler_params=pltpu.CompilerParams(
            use_tc_tiling_on_sc=False,
            needs_layout_passes=False,
        ),
    )


def _pack_body(x_ref, o_ref):
    blk = x_ref.shape[0]
    half = blk // 2
    xp = jnp.pad(x_ref[:, :], ((0, 0), (0, 256 - HIST)))
    x00 = lax.slice(xp, (0, 0), (half, 128))
    x01 = lax.slice(xp, (0, 128), (half, 256))
    x10 = lax.slice(xp, (half, 0), (blk, 128))
    x11 = lax.slice(xp, (half, 128), (blk, 256))
    w = x00 | (x01 << 8) | (x10 << 16) | (x11 << 24)
    o_ref[...] = w.reshape(-1)


def _pack(x):
    blk = 2048
    return pl.pallas_call(
        _pack_body,
        grid=(BATCH // blk,),
        in_specs=[pl.BlockSpec((blk, HIST), lambda i: (i, 0))],
        out_specs=pl.BlockSpec((blk * 64,), lambda i: (i,)),
        out_shape=jax.ShapeDtypeStruct((BATCH * 64,), jnp.int32),
    )(x)


def _matmul_body(c_ref, t_ref, o_ref):
    o_ref[:, :] = jnp.tanh(
        jnp.dot(
            c_ref[:, :],
            t_ref[:, :],
            preferred_element_type=jnp.float32,
            precision=lax.Precision.HIGHEST,
        )
    )


def _matmul_tanh(counts, table128):
    blk = 8192
    return pl.pallas_call(
        _matmul_body,
        grid=(BATCH // blk,),
        in_specs=[
            pl.BlockSpec((blk, VPAD), lambda i: (i, 0)),
            pl.BlockSpec((VPAD, 128), lambda i: (0, 0)),
        ],
        out_specs=pl.BlockSpec((blk, 128), lambda i: (i, 0)),
        out_shape=jax.ShapeDtypeStruct((BATCH, 128), jnp.float32),
    )(counts, table128)


def kernel(x, table):
    x_pk = _pack(x)
    counts = _make_hist()(x_pk)
    table128 = jnp.concatenate([table, jnp.zeros((105, 128), table.dtype)], axis=0)
    return _matmul_tanh(counts, table128)
